# Initial kernel scaffold; baseline (speedup 1.0000x reference)
import jax
import jax.numpy as jnp
from jax.experimental import pallas as pl


def kernel(bag_label, h, A, W, b):
    raise NotImplementedError("write your pallas kernel here")


# speedup vs baseline: 3.6655x; 3.6655x over previous
import functools

import jax
import jax.numpy as jnp
from jax import lax
from jax.experimental import pallas as pl
from jax.experimental.pallas import tpu as pltpu
from jax.experimental.pallas import tpu_sc as plsc

N = 100000
DIM = 512
N_INS = 8
NSUB = 16
CHUNK = N // NSUB
NV_FULL = CHUNK // 16
TAIL = CHUNK - NV_FULL * 16
NEG_INF = float("-inf")


def _merge_top16(t_k, t_i, in_k_desc, in_i_desc):
    m = in_k_desc > t_k
    k2 = jnp.where(m, in_k_desc, t_k)
    i2 = jnp.where(m, in_i_desc, t_i)
    k3, i3 = plsc.sort_key_val(k2, i2)
    return k3, i3


def _sc_body(a_hbm, h_hbm, wt_hbm, b_hbm, bl_hbm, out_lu, out_pr,
             a_v, wt_v, b_v, bl_v, kv_stage, iv_stage,
             cand_k_sh, cand_i_sh, allk_v, alli_v, idxbuf_v,
             hrow_buf, hrows_sh, myrow_v, res_v, out_stage_sh, st_v,
             lu_v, pr_v, sem):
    cid = lax.axis_index("c")
    sid = lax.axis_index("s")
    lane = lax.iota(jnp.int32, 16)

    pltpu.sync_copy(a_hbm.at[sid], a_v)
    pltpu.sync_copy(wt_hbm, wt_v)
    pltpu.sync_copy(b_hbm, b_v)
    pltpu.sync_copy(bl_hbm, bl_v)

    col = bl_v[...]
    sigma = 1.0 - 2.0 * cid.astype(jnp.float32)
    base = sid * CHUNK

    t0_k = jnp.full((16,), NEG_INF, dtype=jnp.float32)
    t0_i = jnp.zeros((16,), dtype=jnp.int32)

    def body(j, carry):
        t_k, t_i = carry
        row = j * 16 + lane
        v = plsc.load_gather(a_v, [row * 2 + col])
        key = v * sigma
        gi = base + row
        ks, gis = plsc.sort_key_val(key, gi, descending=True)
        return _merge_top16(t_k, t_i, ks, gis)

    t_k, t_i = lax.fori_loop(0, NV_FULL, body, (t0_k, t0_i))

    row = NV_FULL * 16 + lane
    valid = lane < TAIL
    rowc = jnp.where(valid, row, 0)
    v = plsc.load_gather(a_v, [rowc * 2 + col])
    key = jnp.where(valid, v * sigma, NEG_INF)
    gi = base + row
    ks, gis = plsc.sort_key_val(key, gi, descending=True)
    t_k, t_i = _merge_top16(t_k, t_i, ks, gis)

    kv_stage[...] = lax.rev(t_k, (0,))
    iv_stage[...] = lax.rev(t_i, (0,))
    pltpu.sync_copy(kv_stage, cand_k_sh.at[pl.ds(sid * 16, 16)])
    pltpu.sync_copy(iv_stage, cand_i_sh.at[pl.ds(sid * 16, 16)])
    plsc.subcore_barrier()

    @pl.when(sid == 0)
    def _phase_b():
        pltpu.sync_copy(cand_k_sh, allk_v)
        pltpu.sync_copy(cand_i_sh, alli_v)
        g_k = jnp.full((16,), NEG_INF, dtype=jnp.float32)
        g_i = jnp.zeros((16,), dtype=jnp.int32)
        for s in range(NSUB):
            g_k, g_i = _merge_top16(g_k, g_i, allk_v[pl.ds(s * 16, 16)],
                                    alli_v[pl.ds(s * 16, 16)])
        idxbuf_v[...] = lax.rev(g_i, (0,))
        pltpu.async_copy(h_hbm.at[idxbuf_v.at[pl.ds(0, N_INS)]],
                         hrow_buf, sem).wait()
        pltpu.sync_copy(hrow_buf, hrows_sh)

    plsc.subcore_barrier()

    inst = lax.rem(sid, N_INS)
    pltpu.sync_copy(hrows_sh.at[inst], myrow_v)
    acc0 = jnp.zeros((16,), dtype=jnp.float32)
    acc1 = jnp.zeros((16,), dtype=jnp.float32)
    for k in range(DIM // 16):
        r = myrow_v[pl.ds(k * 16, 16)]
        acc0 = acc0 + r * wt_v[pl.ds(k * 16, 16)]
        acc1 = acc1 + r * wt_v[pl.ds(DIM + k * 16, 16)]
    bv = b_v[...]
    l0 = jnp.sum(acc0) + bv[0]
    l1 = jnp.sum(acc1) + bv[1]
    m = jnp.maximum(l0, l1)
    e0 = jnp.exp(jnp.full((16,), l0 - m, dtype=jnp.float32))
    e1 = jnp.exp(jnp.full((16,), l1 - m, dtype=jnp.float32))
    ssum = e0 + e1
    p0 = e0 / ssum
    p1 = e1 / ssum
    res = jnp.where(lane == 0, l0,
          jnp.where(lane == 1, l1,
          jnp.where(lane == 2, p0,
          jnp.where(lane == 3, p1, 0.0))))
    res_v[...] = res
    pltpu.sync_copy(res_v, out_stage_sh.at[pl.ds(sid * 16, 16)])
    plsc.subcore_barrier()

    @pl.when(sid == 0)
    def _phase_d():
        pltpu.sync_copy(out_stage_sh, st_v)
        pos = lax.shift_right_logical(lane, 1) * 16 + lax.bitwise_and(lane, 1)
        lu_v[...] = plsc.load_gather(st_v, [pos])
        pr_v[...] = plsc.load_gather(st_v, [pos + 2])
        half = pl.multiple_of(cid * 16, 16)
        pltpu.sync_copy(lu_v, out_lu.at[pl.ds(half, 16)])
        pltpu.sync_copy(pr_v, out_pr.at[pl.ds(half, 16)])


@jax.jit
def _sc_call(a3, h, wt, b, bl):
    f32 = jnp.float32
    i32 = jnp.int32
    run = pl.kernel(
        _sc_body,
        out_type=(
            jax.ShapeDtypeStruct((4 * N_INS,), f32),
            jax.ShapeDtypeStruct((4 * N_INS,), f32),
        ),
        mesh=plsc.VectorSubcoreMesh(core_axis_name="c", subcore_axis_name="s"),
        compiler_params=pltpu.CompilerParams(needs_layout_passes=False),
        scratch_types=[
            pltpu.VMEM((2 * CHUNK,), f32),
            pltpu.VMEM((2 * DIM,), f32),
            pltpu.VMEM((16,), f32),
            pltpu.VMEM((16,), i32),
            pltpu.VMEM((16,), f32),
            pltpu.VMEM((16,), i32),
            pltpu.VMEM_SHARED((NSUB * 16,), f32),
            pltpu.VMEM_SHARED((NSUB * 16,), i32),
            pltpu.VMEM((NSUB * 16,), f32),
            pltpu.VMEM((NSUB * 16,), i32),
            pltpu.VMEM((16,), i32),
            pltpu.VMEM((N_INS, DIM), f32),
            pltpu.VMEM_SHARED((N_INS, DIM), f32),
            pltpu.VMEM((DIM,), f32),
            pltpu.VMEM((16,), f32),
            pltpu.VMEM_SHARED((NSUB * 16,), f32),
            pltpu.VMEM((NSUB * 16,), f32),
            pltpu.VMEM((16,), f32),
            pltpu.VMEM((16,), f32),
            pltpu.SemaphoreType.DMA,
        ],
    )
    return run(a3, h, wt, b, bl)


def kernel(bag_label, h, A, W, b):
    a3 = A.reshape(NSUB, 2 * CHUNK)
    wt = W.T.reshape(-1)
    bl = jnp.full((16,), bag_label, dtype=jnp.int32)
    b16 = jnp.zeros((16,), jnp.float32).at[:2].set(b)
    lu_flat, pr_flat = _sc_call(a3, h, wt, b16, bl)
    logits_unnorm = lu_flat.reshape(2 * N_INS, 2)
    logits = pr_flat.reshape(2 * N_INS, 2)
    ins_labels = jnp.concatenate(
        [jnp.ones((N_INS,), jnp.int32), jnp.zeros((N_INS,), jnp.int32)])
    return (ins_labels, logits_unnorm, logits)


# speedup vs baseline: 3.8186x; 1.0418x over previous
import jax
import jax.numpy as jnp
from jax import lax
from jax.experimental import pallas as pl
from jax.experimental.pallas import tpu as pltpu
from jax.experimental.pallas import tpu_sc as plsc

N = 100000
DIM = 512
N_INS = 8
NSUB = 16
CHUNK = N // NSUB
NV = CHUNK // 16
BLK = 8
NBLK = NV // BLK
REM = NV - NBLK * BLK
TAIL = CHUNK - NV * 16
NEG_INF = float("-inf")


def _merge_top16(t_k, t_i, in_k_desc, in_i_desc):
    m = in_k_desc > t_k
    k3, i3 = plsc.sort_key_val(jnp.where(m, in_k_desc, t_k),
                               jnp.where(m, in_i_desc, t_i))
    return k3, i3


def _merge_pair(ka, ia, kb, ib, descending):
    m = kb > ka
    k3, i3 = plsc.sort_key_val(jnp.where(m, kb, ka), jnp.where(m, ib, ia),
                               descending=descending)
    return k3, i3


def _sc_body(a_hbm, h_hbm, wt_hbm, b_hbm, bl_hbm, out_lu, out_pr,
             a_v, wt_v, b_v, bl_v, kv_stage, iv_stage,
             cand_k_sh, cand_i_sh, allk_v, alli_v, idxbuf_v,
             hrow_v, lu_v, ex_v, pr_v, sem):
    cid = lax.axis_index("c")
    sid = lax.axis_index("s")
    lane = lax.iota(jnp.int32, 16)

    a_dma = pltpu.async_copy(a_hbm.at[sid], a_v, sem)
    pltpu.sync_copy(bl_hbm, bl_v)

    col = bl_v[...]
    sigma = 1.0 - 2.0 * cid.astype(jnp.float32)
    base = sid * CHUNK
    a_dma.wait()

    def load_sorted(row, descending):
        v = plsc.load_gather(a_v, [row * 2 + col])
        return plsc.sort_key_val(v * sigma, base + row, descending=descending)

    t0_k = jnp.full((16,), NEG_INF, dtype=jnp.float32)
    t0_i = jnp.zeros((16,), dtype=jnp.int32)

    def body(j, carry):
        t_k, t_i = carry
        leaves = [load_sorted(j * (BLK * 16) + u * 16 + lane, u % 2 == 1)
                  for u in range(BLK)]
        l1 = [_merge_pair(*leaves[2 * p], *leaves[2 * p + 1], p % 2 == 1)
              for p in range(4)]
        l2 = [_merge_pair(*l1[2 * p], *l1[2 * p + 1], p == 1)
              for p in range(2)]
        bk, bi = _merge_pair(*l2[0], *l2[1], True)
        return _merge_top16(t_k, t_i, bk, bi)

    t_k, t_i = lax.fori_loop(0, NBLK, body, (t0_k, t0_i))

    for u in range(REM):
        ks, gis = load_sorted((NBLK * BLK + u) * 16 + lane, True)
        t_k, t_i = _merge_top16(t_k, t_i, ks, gis)

    row = NV * 16 + lane
    valid = lane < TAIL
    v = plsc.load_gather(a_v, [jnp.where(valid, row, 0) * 2 + col])
    ks, gis = plsc.sort_key_val(jnp.where(valid, v * sigma, NEG_INF),
                                base + row, descending=True)
    t_k, t_i = _merge_top16(t_k, t_i, ks, gis)

    kv_stage[...] = lax.rev(t_k, (0,))
    iv_stage[...] = lax.rev(t_i, (0,))
    pltpu.sync_copy(kv_stage, cand_k_sh.at[pl.ds(sid * 16, 16)])
    pltpu.sync_copy(iv_stage, cand_i_sh.at[pl.ds(sid * 16, 16)])
    plsc.subcore_barrier()

    @pl.when(sid == 0)
    def _phase_b():
        pltpu.sync_copy(cand_k_sh, allk_v)
        pltpu.sync_copy(cand_i_sh, alli_v)
        pltpu.sync_copy(wt_hbm, wt_v)
        pltpu.sync_copy(b_hbm, b_v)

        def cand(s, descending):
            kk = allk_v[pl.ds(s * 16, 16)]
            ii = alli_v[pl.ds(s * 16, 16)]
            if descending:
                return kk, ii
            return lax.rev(kk, (0,)), lax.rev(ii, (0,))

        lvl = [cand(s, s % 2 == 1) for s in range(NSUB)]
        while len(lvl) > 2:
            lvl = [_merge_pair(*lvl[2 * p], *lvl[2 * p + 1], p % 2 == 1)
                   for p in range(len(lvl) // 2)]
        g_k, g_i = _merge_pair(*lvl[0], *lvl[1], False)

        idxbuf_v[...] = lax.rev(g_i, (0,))
        pltpu.async_copy(h_hbm.at[idxbuf_v.at[pl.ds(0, N_INS)]],
                         hrow_v, sem).wait()

        bv = b_v[...]
        lu = jnp.zeros((16,), dtype=jnp.float32)
        for i in range(N_INS):
            acc0 = jnp.zeros((16,), dtype=jnp.float32)
            acc1 = jnp.zeros((16,), dtype=jnp.float32)
            for k in range(DIM // 16):
                r = hrow_v[i, pl.ds(k * 16, 16)]
                acc0 = acc0 + r * wt_v[pl.ds(k * 16, 16)]
                acc1 = acc1 + r * wt_v[pl.ds(DIM + k * 16, 16)]
            l0 = jnp.sum(acc0) + bv[0]
            l1 = jnp.sum(acc1) + bv[1]
            lu = jnp.where(lane == 2 * i, l0, lu)
            lu = jnp.where(lane == 2 * i + 1, l1, lu)

        lu_v[...] = lu
        partner = lax.bitwise_xor(lane, 1)
        mx = jnp.maximum(lu, plsc.load_gather(lu_v, [partner]))
        ex = jnp.exp(lu - mx)
        ex_v[...] = ex
        pr = ex / (ex + plsc.load_gather(ex_v, [partner]))
        pr_v[...] = pr

        half = pl.multiple_of(cid * 16, 16)
        pltpu.sync_copy(lu_v, out_lu.at[pl.ds(half, 16)])
        pltpu.sync_copy(pr_v, out_pr.at[pl.ds(half, 16)])


@jax.jit
def _sc_call(a3, h, wt, b, bl):
    f32 = jnp.float32
    i32 = jnp.int32
    run = pl.kernel(
        _sc_body,
        out_type=(
            jax.ShapeDtypeStruct((4 * N_INS,), f32),
            jax.ShapeDtypeStruct((4 * N_INS,), f32),
        ),
        mesh=plsc.VectorSubcoreMesh(core_axis_name="c", subcore_axis_name="s"),
        compiler_params=pltpu.CompilerParams(needs_layout_passes=False),
        scratch_types=[
            pltpu.VMEM((2 * CHUNK,), f32),
            pltpu.VMEM((2 * DIM,), f32),
            pltpu.VMEM((16,), f32),
            pltpu.VMEM((16,), i32),
            pltpu.VMEM((16,), f32),
            pltpu.VMEM((16,), i32),
            pltpu.VMEM_SHARED((NSUB * 16,), f32),
            pltpu.VMEM_SHARED((NSUB * 16,), i32),
            pltpu.VMEM((NSUB * 16,), f32),
            pltpu.VMEM((NSUB * 16,), i32),
            pltpu.VMEM((16,), i32),
            pltpu.VMEM((N_INS, DIM), f32),
            pltpu.VMEM((16,), f32),
            pltpu.VMEM((16,), f32),
            pltpu.VMEM((16,), f32),
            pltpu.SemaphoreType.DMA,
        ],
    )
    return run(a3, h, wt, b, bl)


def kernel(bag_label, h, A, W, b):
    a3 = A.reshape(NSUB, 2 * CHUNK)
    wt = W.T.reshape(-1)
    bl = jnp.full((16,), bag_label, dtype=jnp.int32)
    b16 = jnp.zeros((16,), jnp.float32).at[:2].set(b)
    lu_flat, pr_flat = _sc_call(a3, h, wt, b16, bl)
    ins_labels = jnp.concatenate(
        [jnp.ones((N_INS,), jnp.int32), jnp.zeros((N_INS,), jnp.int32)])
    return (ins_labels, lu_flat.reshape(2 * N_INS, 2),
            pr_flat.reshape(2 * N_INS, 2))


# speedup vs baseline: 10.0686x; 2.6368x over previous
import jax
import jax.numpy as jnp
from jax import lax
from jax.experimental import pallas as pl
from jax.experimental.pallas import tpu as pltpu
from jax.experimental.pallas import tpu_sc as plsc

N = 100000
NPAD = 100096
DIM = 512
N_INS = 8
NSUB = 16
CHUNK = NPAD // NSUB
NV = CHUNK // 16
BLK = 8
NBLK = NV // BLK
REM = NV - NBLK * BLK
NEG_INF = float("-inf")


def _merge_top16(t_k, t_i, in_k_desc, in_i_desc):
    m = in_k_desc > t_k
    k3, i3 = plsc.sort_key_val(jnp.where(m, in_k_desc, t_k),
                               jnp.where(m, in_i_desc, t_i))
    return k3, i3


def _merge_pair(ka, ia, kb, ib, descending):
    m = kb > ka
    k3, i3 = plsc.sort_key_val(jnp.where(m, kb, ka), jnp.where(m, ib, ia),
                               descending=descending)
    return k3, i3


def _sc_body(a_hbm, h_hbm, wt_hbm, b_hbm, out_lu, out_pr,
             a_v, wt_v, b_v, kv_stage, iv_stage,
             cand_k_sh, cand_i_sh, allk_v, alli_v, idxbuf_v,
             hrow_v, lu_v, ex_v, pr_v, sem):
    cid = lax.axis_index("c")
    sid = lax.axis_index("s")
    lane = lax.iota(jnp.int32, 16)

    base = sid * CHUNK
    pltpu.sync_copy(a_hbm.at[pl.ds(base, CHUNK)], a_v)
    sigma = 1.0 - 2.0 * cid.astype(jnp.float32)

    def load_sorted(off, descending):
        v = a_v[pl.ds(off, 16)]
        gi = base + off + lane
        key = jnp.where(gi < N, v * sigma, NEG_INF)
        return plsc.sort_key_val(key, gi, descending=descending)

    t0_k = jnp.full((16,), NEG_INF, dtype=jnp.float32)
    t0_i = jnp.zeros((16,), dtype=jnp.int32)

    def body(j, carry):
        t_k, t_i = carry
        leaves = [load_sorted(j * (BLK * 16) + u * 16, u % 2 == 1)
                  for u in range(BLK)]
        l1 = [_merge_pair(*leaves[2 * p], *leaves[2 * p + 1], p % 2 == 1)
              for p in range(4)]
        l2 = [_merge_pair(*l1[2 * p], *l1[2 * p + 1], p == 1)
              for p in range(2)]
        bk, bi = _merge_pair(*l2[0], *l2[1], True)
        return _merge_top16(t_k, t_i, bk, bi)

    t_k, t_i = lax.fori_loop(0, NBLK, body, (t0_k, t0_i))

    for u in range(REM):
        ks, gis = load_sorted((NBLK * BLK + u) * 16, True)
        t_k, t_i = _merge_top16(t_k, t_i, ks, gis)

    kv_stage[...] = lax.rev(t_k, (0,))
    iv_stage[...] = lax.rev(t_i, (0,))
    pltpu.sync_copy(kv_stage, cand_k_sh.at[pl.ds(sid * 16, 16)])
    pltpu.sync_copy(iv_stage, cand_i_sh.at[pl.ds(sid * 16, 16)])
    plsc.subcore_barrier()

    @pl.when(sid == 0)
    def _phase_b():
        pltpu.sync_copy(cand_k_sh, allk_v)
        pltpu.sync_copy(cand_i_sh, alli_v)
        pltpu.sync_copy(wt_hbm, wt_v)
        pltpu.sync_copy(b_hbm, b_v)

        def cand_set(s, descending):
            kk = allk_v[pl.ds(s * 16, 16)]
            ii = alli_v[pl.ds(s * 16, 16)]
            if descending:
                return kk, ii
            return lax.rev(kk, (0,)), lax.rev(ii, (0,))

        lvl = [cand_set(s, s % 2 == 1) for s in range(NSUB)]
        while len(lvl) > 2:
            lvl = [_merge_pair(*lvl[2 * p], *lvl[2 * p + 1], p % 2 == 1)
                   for p in range(len(lvl) // 2)]
        g_k, g_i = _merge_pair(*lvl[0], *lvl[1], False)

        idxbuf_v[...] = lax.rev(g_i, (0,))
        pltpu.async_copy(h_hbm.at[idxbuf_v.at[pl.ds(0, N_INS)]],
                         hrow_v, sem).wait()

        bv = b_v[...]
        lu = jnp.zeros((16,), dtype=jnp.float32)
        for i in range(N_INS):
            acc0 = jnp.zeros((16,), dtype=jnp.float32)
            acc1 = jnp.zeros((16,), dtype=jnp.float32)
            for k in range(DIM // 16):
                r = hrow_v[i, pl.ds(k * 16, 16)]
                acc0 = acc0 + r * wt_v[pl.ds(k * 16, 16)]
                acc1 = acc1 + r * wt_v[pl.ds(DIM + k * 16, 16)]
            l0 = jnp.sum(acc0) + bv[0]
            l1 = jnp.sum(acc1) + bv[1]
            lu = jnp.where(lane == 2 * i, l0, lu)
            lu = jnp.where(lane == 2 * i + 1, l1, lu)

        lu_v[...] = lu
        partner = lax.bitwise_xor(lane, 1)
        mx = jnp.maximum(lu, plsc.load_gather(lu_v, [partner]))
        ex = jnp.exp(lu - mx)
        ex_v[...] = ex
        pr = ex / (ex + plsc.load_gather(ex_v, [partner]))
        pr_v[...] = pr

        half = pl.multiple_of(cid * 16, 16)
        pltpu.sync_copy(lu_v, out_lu.at[pl.ds(half, 16)])
        pltpu.sync_copy(pr_v, out_pr.at[pl.ds(half, 16)])


@jax.jit
def _sc_call(a_col, h, wt, b):
    f32 = jnp.float32
    i32 = jnp.int32
    run = pl.kernel(
        _sc_body,
        out_type=(
            jax.ShapeDtypeStruct((4 * N_INS,), f32),
            jax.ShapeDtypeStruct((4 * N_INS,), f32),
        ),
        mesh=plsc.VectorSubcoreMesh(core_axis_name="c", subcore_axis_name="s"),
        compiler_params=pltpu.CompilerParams(needs_layout_passes=False),
        scratch_types=[
            pltpu.VMEM((CHUNK,), f32),
            pltpu.VMEM((2 * DIM,), f32),
            pltpu.VMEM((16,), f32),
            pltpu.VMEM((16,), f32),
            pltpu.VMEM((16,), i32),
            pltpu.VMEM_SHARED((NSUB * 16,), f32),
            pltpu.VMEM_SHARED((NSUB * 16,), i32),
            pltpu.VMEM((NSUB * 16,), f32),
            pltpu.VMEM((NSUB * 16,), i32),
            pltpu.VMEM((16,), i32),
            pltpu.VMEM((N_INS, DIM), f32),
            pltpu.VMEM((16,), f32),
            pltpu.VMEM((16,), f32),
            pltpu.VMEM((16,), f32),
            pltpu.SemaphoreType.DMA,
        ],
    )
    return run(a_col, h, wt, b)


def kernel(bag_label, h, A, W, b):
    bl = jnp.asarray(bag_label, jnp.int32)
    a_col = jnp.where(bl == 0, A[:, 0, 0], A[:, 0, 1])
    a_pad = jnp.concatenate([a_col, jnp.zeros((NPAD - N,), jnp.float32)])
    wt = W.T.reshape(-1)
    b16 = jnp.zeros((16,), jnp.float32).at[:2].set(b)
    lu_flat, pr_flat = _sc_call(a_pad, h, wt, b16)
    ins_labels = jnp.concatenate(
        [jnp.ones((N_INS,), jnp.int32), jnp.zeros((N_INS,), jnp.int32)])
    return (ins_labels, lu_flat.reshape(2 * N_INS, 2),
            pr_flat.reshape(2 * N_INS, 2))


# speedup vs baseline: 10.3021x; 1.0232x over previous
import jax
import jax.numpy as jnp
from jax import lax
from jax.experimental import pallas as pl
from jax.experimental.pallas import tpu as pltpu
from jax.experimental.pallas import tpu_sc as plsc

N = 100000
NPAD = 100096
DIM = 512
N_INS = 8
NSUB = 16
CHUNK = NPAD // NSUB
NV = CHUNK // 16
BLK = 8
NBLK = NV // BLK
REM = NV - NBLK * BLK
NEG_INF = float("-inf")


def _merge_top16(t_k, t_i, in_k_desc, in_i_desc):
    m = in_k_desc > t_k
    k3, i3 = plsc.sort_key_val(jnp.where(m, in_k_desc, t_k),
                               jnp.where(m, in_i_desc, t_i))
    return k3, i3


def _merge_pair(ka, ia, kb, ib, descending):
    m = kb > ka
    k3, i3 = plsc.sort_key_val(jnp.where(m, kb, ka), jnp.where(m, ib, ia),
                               descending=descending)
    return k3, i3


def _sc_body(a_hbm, h_hbm, wb_hbm, out_all,
             a_v, wt_v, b_v, kv_stage, iv_stage,
             cand_k_sh, cand_i_sh, allk_v, alli_v, idxbuf_v,
             hrow_v, lu_v, ex_v, pr_v, sem):
    cid = lax.axis_index("c")
    sid = lax.axis_index("s")
    lane = lax.iota(jnp.int32, 16)

    base = sid * CHUNK
    pltpu.sync_copy(a_hbm.at[pl.ds(base, CHUNK)], a_v)
    sigma = 1.0 - 2.0 * cid.astype(jnp.float32)

    def load_sorted(off, descending):
        v = a_v[pl.ds(off, 16)]
        gi = base + off + lane
        key = jnp.where(gi < N, v * sigma, NEG_INF)
        return plsc.sort_key_val(key, gi, descending=descending)

    t0_k = jnp.full((16,), NEG_INF, dtype=jnp.float32)
    t0_i = jnp.zeros((16,), dtype=jnp.int32)

    def body(j, carry):
        t_k, t_i = carry
        leaves = [load_sorted(j * (BLK * 16) + u * 16, u % 2 == 1)
                  for u in range(BLK)]
        l1 = [_merge_pair(*leaves[2 * p], *leaves[2 * p + 1], p % 2 == 1)
              for p in range(4)]
        l2 = [_merge_pair(*l1[2 * p], *l1[2 * p + 1], p == 1)
              for p in range(2)]
        bk, bi = _merge_pair(*l2[0], *l2[1], True)
        return _merge_top16(t_k, t_i, bk, bi)

    t_k, t_i = lax.fori_loop(0, NBLK, body, (t0_k, t0_i))

    for u in range(REM):
        ks, gis = load_sorted((NBLK * BLK + u) * 16, True)
        t_k, t_i = _merge_top16(t_k, t_i, ks, gis)

    kv_stage[...] = lax.rev(t_k, (0,))
    iv_stage[...] = lax.rev(t_i, (0,))
    pltpu.sync_copy(kv_stage, cand_k_sh.at[pl.ds(sid * 16, 16)])
    pltpu.sync_copy(iv_stage, cand_i_sh.at[pl.ds(sid * 16, 16)])
    plsc.subcore_barrier()

    @pl.when(sid == 0)
    def _phase_b():
        pltpu.sync_copy(cand_k_sh, allk_v)
        pltpu.sync_copy(cand_i_sh, alli_v)
        pltpu.sync_copy(wb_hbm.at[pl.ds(0, 2 * DIM)], wt_v)
        pltpu.sync_copy(wb_hbm.at[pl.ds(2 * DIM, 16)], b_v)

        def cand_set(s, descending):
            kk = allk_v[pl.ds(s * 16, 16)]
            ii = alli_v[pl.ds(s * 16, 16)]
            if descending:
                return kk, ii
            return lax.rev(kk, (0,)), lax.rev(ii, (0,))

        lvl = [cand_set(s, s % 2 == 1) for s in range(NSUB)]
        while len(lvl) > 2:
            lvl = [_merge_pair(*lvl[2 * p], *lvl[2 * p + 1], p % 2 == 1)
                   for p in range(len(lvl) // 2)]
        g_k, g_i = _merge_pair(*lvl[0], *lvl[1], False)

        idxbuf_v[...] = lax.rev(g_i, (0,))
        pltpu.async_copy(h_hbm.at[idxbuf_v.at[pl.ds(0, N_INS)]],
                         hrow_v, sem).wait()

        bv = b_v[...]

        def inst_body(i, lu):
            def k_body(k, accs):
                acc0, acc1 = accs
                r = hrow_v[i, pl.ds(k * 16, 16)]
                acc0 = acc0 + r * wt_v[pl.ds(k * 16, 16)]
                acc1 = acc1 + r * wt_v[pl.ds(DIM + k * 16, 16)]
                return (acc0, acc1)

            z = jnp.zeros((16,), dtype=jnp.float32)
            acc0, acc1 = lax.fori_loop(0, DIM // 16, k_body, (z, z))
            l0 = jnp.sum(acc0) + bv[0]
            l1 = jnp.sum(acc1) + bv[1]
            lu = jnp.where(lane == 2 * i, l0, lu)
            return jnp.where(lane == 2 * i + 1, l1, lu)

        lu = lax.fori_loop(0, N_INS, inst_body, jnp.zeros((16,), jnp.float32))

        lu_v[...] = lu
        partner = lax.bitwise_xor(lane, 1)
        mx = jnp.maximum(lu, plsc.load_gather(lu_v, [partner]))
        ex = jnp.exp(lu - mx)
        ex_v[...] = ex
        pr = ex / (ex + plsc.load_gather(ex_v, [partner]))
        pr_v[...] = pr

        half = pl.multiple_of(cid * 16, 16)
        pltpu.sync_copy(lu_v, out_all.at[pl.ds(half, 16)])
        pltpu.sync_copy(pr_v, out_all.at[pl.ds(32 + half, 16)])


@jax.jit
def _sc_call(a_col, h, wb):
    f32 = jnp.float32
    i32 = jnp.int32
    run = pl.kernel(
        _sc_body,
        out_type=jax.ShapeDtypeStruct((8 * N_INS,), f32),
        mesh=plsc.VectorSubcoreMesh(core_axis_name="c", subcore_axis_name="s"),
        compiler_params=pltpu.CompilerParams(needs_layout_passes=False),
        scratch_types=[
            pltpu.VMEM((CHUNK,), f32),
            pltpu.VMEM((2 * DIM,), f32),
            pltpu.VMEM((16,), f32),
            pltpu.VMEM((16,), f32),
            pltpu.VMEM((16,), i32),
            pltpu.VMEM_SHARED((NSUB * 16,), f32),
            pltpu.VMEM_SHARED((NSUB * 16,), i32),
            pltpu.VMEM((NSUB * 16,), f32),
            pltpu.VMEM((NSUB * 16,), i32),
            pltpu.VMEM((16,), i32),
            pltpu.VMEM((N_INS, DIM), f32),
            pltpu.VMEM((16,), f32),
            pltpu.VMEM((16,), f32),
            pltpu.VMEM((16,), f32),
            pltpu.SemaphoreType.DMA,
        ],
    )
    return run(a_col, h, wb)


def kernel(bag_label, h, A, W, b):
    bl = jnp.asarray(bag_label, jnp.int32)
    a_col = jnp.where(bl == 0, A[:, 0, 0], A[:, 0, 1])
    a_pad = jnp.concatenate([a_col, jnp.zeros((NPAD - N,), jnp.float32)])
    wb = jnp.concatenate(
        [W.T.reshape(-1), b, jnp.zeros((14,), jnp.float32)])
    out_all = _sc_call(a_pad, h, wb)
    lu_flat, pr_flat = out_all[:32], out_all[32:]
    ins_labels = jnp.concatenate(
        [jnp.ones((N_INS,), jnp.int32), jnp.zeros((N_INS,), jnp.int32)])
    return (ins_labels, lu_flat.reshape(2 * N_INS, 2),
            pr_flat.reshape(2 * N_INS, 2))
